# baseline (speedup 1.0000x reference)
import functools

import numpy as np
import jax
import jax.numpy as jnp
from jax import lax
from jax.experimental import pallas as pl
from jax.experimental.pallas import tpu as pltpu
from jax.experimental.pallas import tpu_sc as plsc

N = 10000
E = 160000
F_IN = 256
H = 512
C = 40
L = 16
ALPHA = 0.1
THETA = 0.5

NCHUNK = 4
CW = H // NCHUNK
NC, NS = 2, 16
B = 125
NB = (E // NS) // B
NPAD = 10240
ROWS_PER_TILE = NPAD // NS
ZR = 16

_ROW_BLK = 2000
_GRID = N // _ROW_BLK


def _sc_mesh():
    return plsc.VectorSubcoreMesh(core_axis_name="c", subcore_axis_name="s",
                                  num_cores=NC, num_subcores=NS)


@functools.partial(
    pl.kernel,
    out_type=jax.ShapeDtypeStruct((NCHUNK, NPAD, CW), jnp.float32),
    mesh=_sc_mesh(),
    scratch_types=[
        pltpu.VMEM_SHARED((NPAD, CW), jnp.float32),
        pltpu.VMEM((NB // 2, B), jnp.int32),
        pltpu.VMEM((NB // 2, B), jnp.int32),
        pltpu.VMEM((B, CW), jnp.float32),
        pltpu.VMEM((B, CW), jnp.float32),
        pltpu.SemaphoreType.DMA,
        pltpu.SemaphoreType.DMA,
        pltpu.SemaphoreType.DMA,
    ],
)
def _sc_segsum(h_ref, x0s_ref, srcoff_ref, dst_ref, out_ref,
               agg, sidx, didx, gbuf0, gbuf1,
               gsem0, gsem1, zsem):
    cid = lax.axis_index("c")
    sid = lax.axis_index("s")
    nh = NB // 2

    for k in range(2):
        chunk = cid * 2 + k
        stripe = pl.ds(sid * ROWS_PER_TILE, ROWS_PER_TILE)
        pltpu.async_copy(x0s_ref.at[chunk, stripe], agg.at[stripe], zsem).wait()
        plsc.subcore_barrier()

        for half in range(2):
            pltpu.sync_copy(srcoff_ref.at[chunk, sid, half], sidx)
            pltpu.sync_copy(dst_ref.at[sid, half], didx)
            pltpu.async_copy(h_ref.at[sidx.at[0]], gbuf0, gsem0)

            def pair(i, carry):
                pltpu.async_copy(h_ref.at[sidx.at[2 * i + 1]], gbuf1, gsem1)
                pltpu.make_async_copy(h_ref.at[sidx.at[0]], gbuf0, gsem0).wait()
                pltpu.sync_copy(gbuf0, agg.at[didx.at[2 * i]], add=True)
                nxt = jnp.minimum(2 * i + 2, nh - 1)
                pltpu.async_copy(h_ref.at[sidx.at[nxt]], gbuf0, gsem0)
                pltpu.make_async_copy(h_ref.at[sidx.at[0]], gbuf1, gsem1).wait()
                pltpu.sync_copy(gbuf1, agg.at[didx.at[2 * i + 1]], add=True)
                return carry
            lax.fori_loop(0, nh // 2, pair, 0)
            pltpu.make_async_copy(h_ref.at[sidx.at[0]], gbuf0, gsem0).wait()
        plsc.subcore_barrier()

        pltpu.sync_copy(agg.at[stripe], out_ref.at[chunk, stripe])


def _lin0_body(x_ref, w_ref, b_ref, o_ref, os_ref):
    t = jnp.dot(x_ref[...], w_ref[...], preferred_element_type=jnp.float32)
    t = jnp.maximum(t + b_ref[...], 0.0)
    for c in range(NCHUNK):
        tc = t[:, c * CW:(c + 1) * CW]
        o_ref[c] = tc
        os_ref[c] = tc * (ALPHA / (1.0 - ALPHA))


def _lin0(x, w, b):
    return pl.pallas_call(
        _lin0_body,
        grid=(_GRID,),
        in_specs=[
            pl.BlockSpec((_ROW_BLK, F_IN), lambda i: (i, 0)),
            pl.BlockSpec((F_IN, H), lambda i: (0, 0)),
            pl.BlockSpec((1, H), lambda i: (0, 0)),
        ],
        out_specs=[pl.BlockSpec((NCHUNK, _ROW_BLK, CW), lambda i: (0, i, 0)),
                   pl.BlockSpec((NCHUNK, _ROW_BLK, CW), lambda i: (0, i, 0))],
        out_shape=[jax.ShapeDtypeStruct((NCHUNK, NPAD, CW), jnp.float32),
                   jax.ShapeDtypeStruct((NCHUNK, NPAD, CW), jnp.float32)],
    )(x, w, b)


def _make_layer_body(beta):
    def body(a_ref, w_ref, o_ref):
        pre = [(1.0 - ALPHA) * a_ref[c] for c in range(NCHUNK)]
        m = jnp.dot(pre[0], w_ref[0], preferred_element_type=jnp.float32)
        for c in range(1, NCHUNK):
            m += jnp.dot(pre[c], w_ref[c], preferred_element_type=jnp.float32)
        for c in range(NCHUNK):
            o_ref[c] = jnp.maximum(
                (1.0 - beta) * pre[c] + beta * m[:, c * CW:(c + 1) * CW], 0.0)
    return body


def _tc_layer(aggc, w4, beta):
    return pl.pallas_call(
        _make_layer_body(beta),
        grid=(_GRID,),
        in_specs=[
            pl.BlockSpec((NCHUNK, _ROW_BLK, CW), lambda i: (0, i, 0)),
            pl.BlockSpec((NCHUNK, CW, H), lambda i: (0, 0, 0)),
        ],
        out_specs=pl.BlockSpec((NCHUNK, _ROW_BLK, CW), lambda i: (0, i, 0)),
        out_shape=jax.ShapeDtypeStruct((NCHUNK, NPAD, CW), jnp.float32),
    )(aggc, w4)


def _make_final_body(beta):
    def body(a_ref, w_ref, w1_ref, b1_ref, o_ref):
        pre = [(1.0 - ALPHA) * a_ref[c] for c in range(NCHUNK)]
        m = jnp.dot(pre[0], w_ref[0], preferred_element_type=jnp.float32)
        for c in range(1, NCHUNK):
            m += jnp.dot(pre[c], w_ref[c], preferred_element_type=jnp.float32)
        o = None
        for c in range(NCHUNK):
            hc = jnp.maximum(
                (1.0 - beta) * pre[c] + beta * m[:, c * CW:(c + 1) * CW], 0.0)
            p = jnp.dot(hc, w1_ref[c], preferred_element_type=jnp.float32)
            o = p if o is None else o + p
        o_ref[...] = o + b1_ref[...]
    return body


def _final_layer(aggc, w4, w1, b1, beta):
    return pl.pallas_call(
        _make_final_body(beta),
        grid=(_GRID,),
        in_specs=[
            pl.BlockSpec((NCHUNK, _ROW_BLK, CW), lambda i: (0, i, 0)),
            pl.BlockSpec((NCHUNK, CW, H), lambda i: (0, 0, 0)),
            pl.BlockSpec((NCHUNK, CW, C), lambda i: (0, 0, 0)),
            pl.BlockSpec((1, C), lambda i: (0, 0)),
        ],
        out_specs=pl.BlockSpec((_ROW_BLK, C), lambda i: (i, 0)),
        out_shape=jax.ShapeDtypeStruct((N, C), jnp.float32),
    )(aggc, w4, w1, b1)


def kernel(x, edge_index, lin0_w, lin0_b, conv_w, lin1_w, lin1_b):
    src = edge_index[0]
    dst = edge_index[1]
    srcoff = (src[None, :]
              + (jnp.arange(NCHUNK, dtype=jnp.int32) * NPAD)[:, None])
    srcoff = srcoff.reshape(NCHUNK, NS, 2, NB // 2, B)
    dst_r = dst.reshape(NS, 2, NB // 2, B)

    x0c, x0s = _lin0(x, lin0_w, lin0_b.reshape(1, H))
    w4 = conv_w.reshape(L, NCHUNK, CW, H)

    h = x0c
    for l in range(L - 1):
        beta = float(np.log(THETA / (l + 1) + 1.0))
        aggc = _sc_segsum(h.reshape(NCHUNK * NPAD, CW), x0s, srcoff, dst_r)
        h = _tc_layer(aggc, w4[l], beta)

    beta = float(np.log(THETA / L + 1.0))
    aggc = _sc_segsum(h.reshape(NCHUNK * NPAD, CW), x0s, srcoff, dst_r)
    return _final_layer(aggc, w4[L - 1],
                        lin1_w.reshape(NCHUNK, CW, C), lin1_b.reshape(1, C),
                        beta)


# speedup vs baseline: 1.0078x; 1.0078x over previous
import functools

import numpy as np
import jax
import jax.numpy as jnp
from jax import lax
from jax.experimental import pallas as pl
from jax.experimental.pallas import tpu as pltpu
from jax.experimental.pallas import tpu_sc as plsc

N = 10000
E = 160000
F_IN = 256
H = 512
C = 40
L = 16
ALPHA = 0.1
THETA = 0.5

NCHUNK = 4
CW = H // NCHUNK
NC, NS = 2, 16
B = 125
NB = (E // NS) // B
NPAD = 10240
ROWS_PER_TILE = NPAD // NS
ZR = 16

_ROW_BLK = 2000
_GRID = N // _ROW_BLK


def _sc_mesh():
    return plsc.VectorSubcoreMesh(core_axis_name="c", subcore_axis_name="s",
                                  num_cores=NC, num_subcores=NS)


@functools.partial(
    pl.kernel,
    out_type=jax.ShapeDtypeStruct((NCHUNK, NPAD, CW), jnp.float32),
    mesh=_sc_mesh(),
    scratch_types=[
        pltpu.VMEM_SHARED((NPAD, CW), jnp.float32),
        pltpu.VMEM((2, NB // 2, B), jnp.int32),
        pltpu.VMEM((B, CW), jnp.float32),
        pltpu.VMEM((B, CW), jnp.float32),
        pltpu.SemaphoreType.DMA,
        pltpu.SemaphoreType.DMA,
        pltpu.SemaphoreType.DMA,
    ],
)
def _sc_segsum(h_ref, x0s_ref, edata_ref, out_ref,
               agg, idxb, gbuf0, gbuf1,
               gsem0, gsem1, zsem):
    cid = lax.axis_index("c")
    sid = lax.axis_index("s")
    nh = NB // 2

    for k in range(2):
        chunk = cid * 2 + k
        stripe = pl.ds(sid * ROWS_PER_TILE, ROWS_PER_TILE)
        pltpu.async_copy(x0s_ref.at[chunk, stripe], agg.at[stripe], zsem).wait()
        plsc.subcore_barrier()

        for half in range(2):
            pltpu.sync_copy(edata_ref.at[chunk, sid, half], idxb)
            pltpu.async_copy(h_ref.at[idxb.at[0, 0]], gbuf0, gsem0)

            def pair(i, carry):
                pltpu.async_copy(h_ref.at[idxb.at[0, 2 * i + 1]], gbuf1, gsem1)
                pltpu.make_async_copy(h_ref.at[idxb.at[0, 0]], gbuf0, gsem0).wait()
                pltpu.sync_copy(gbuf0, agg.at[idxb.at[1, 2 * i]], add=True)
                nxt = jnp.minimum(2 * i + 2, nh - 1)
                pltpu.async_copy(h_ref.at[idxb.at[0, nxt]], gbuf0, gsem0)
                pltpu.make_async_copy(h_ref.at[idxb.at[0, 0]], gbuf1, gsem1).wait()
                pltpu.sync_copy(gbuf1, agg.at[idxb.at[1, 2 * i + 1]], add=True)
                return carry
            lax.fori_loop(0, nh // 2, pair, 0)
            pltpu.make_async_copy(h_ref.at[idxb.at[0, 0]], gbuf0, gsem0).wait()
        plsc.subcore_barrier()

        pltpu.sync_copy(agg.at[stripe], out_ref.at[chunk, stripe])


def _lin0_body(x_ref, w_ref, b_ref, o_ref, os_ref):
    t = jnp.dot(x_ref[...], w_ref[...], preferred_element_type=jnp.float32)
    t = jnp.maximum(t + b_ref[...], 0.0)
    for c in range(NCHUNK):
        tc = t[:, c * CW:(c + 1) * CW]
        o_ref[c] = tc
        os_ref[c] = tc * (ALPHA / (1.0 - ALPHA))


def _lin0(x, w, b):
    return pl.pallas_call(
        _lin0_body,
        grid=(_GRID,),
        in_specs=[
            pl.BlockSpec((_ROW_BLK, F_IN), lambda i: (i, 0)),
            pl.BlockSpec((F_IN, H), lambda i: (0, 0)),
            pl.BlockSpec((1, H), lambda i: (0, 0)),
        ],
        out_specs=[pl.BlockSpec((NCHUNK, _ROW_BLK, CW), lambda i: (0, i, 0)),
                   pl.BlockSpec((NCHUNK, _ROW_BLK, CW), lambda i: (0, i, 0))],
        out_shape=[jax.ShapeDtypeStruct((NCHUNK, NPAD, CW), jnp.float32),
                   jax.ShapeDtypeStruct((NCHUNK, NPAD, CW), jnp.float32)],
    )(x, w, b)


def _make_layer_body(beta):
    def body(a_ref, w_ref, o_ref):
        pre = [(1.0 - ALPHA) * a_ref[c] for c in range(NCHUNK)]
        m = jnp.dot(pre[0], w_ref[0], preferred_element_type=jnp.float32)
        for c in range(1, NCHUNK):
            m += jnp.dot(pre[c], w_ref[c], preferred_element_type=jnp.float32)
        for c in range(NCHUNK):
            o_ref[c] = jnp.maximum(
                (1.0 - beta) * pre[c] + beta * m[:, c * CW:(c + 1) * CW], 0.0)
    return body


def _tc_layer(aggc, w4, beta):
    return pl.pallas_call(
        _make_layer_body(beta),
        grid=(_GRID,),
        in_specs=[
            pl.BlockSpec((NCHUNK, _ROW_BLK, CW), lambda i: (0, i, 0)),
            pl.BlockSpec((NCHUNK, CW, H), lambda i: (0, 0, 0)),
        ],
        out_specs=pl.BlockSpec((NCHUNK, _ROW_BLK, CW), lambda i: (0, i, 0)),
        out_shape=jax.ShapeDtypeStruct((NCHUNK, NPAD, CW), jnp.float32),
    )(aggc, w4)


def _make_final_body(beta):
    def body(a_ref, w_ref, w1_ref, b1_ref, o_ref):
        pre = [(1.0 - ALPHA) * a_ref[c] for c in range(NCHUNK)]
        m = jnp.dot(pre[0], w_ref[0], preferred_element_type=jnp.float32)
        for c in range(1, NCHUNK):
            m += jnp.dot(pre[c], w_ref[c], preferred_element_type=jnp.float32)
        o = None
        for c in range(NCHUNK):
            hc = jnp.maximum(
                (1.0 - beta) * pre[c] + beta * m[:, c * CW:(c + 1) * CW], 0.0)
            p = jnp.dot(hc, w1_ref[c], preferred_element_type=jnp.float32)
            o = p if o is None else o + p
        o_ref[...] = o + b1_ref[...]
    return body


def _final_layer(aggc, w4, w1, b1, beta):
    return pl.pallas_call(
        _make_final_body(beta),
        grid=(_GRID,),
        in_specs=[
            pl.BlockSpec((NCHUNK, _ROW_BLK, CW), lambda i: (0, i, 0)),
            pl.BlockSpec((NCHUNK, CW, H), lambda i: (0, 0, 0)),
            pl.BlockSpec((NCHUNK, CW, C), lambda i: (0, 0, 0)),
            pl.BlockSpec((1, C), lambda i: (0, 0)),
        ],
        out_specs=pl.BlockSpec((_ROW_BLK, C), lambda i: (i, 0)),
        out_shape=jax.ShapeDtypeStruct((N, C), jnp.float32),
    )(aggc, w4, w1, b1)


def kernel(x, edge_index, lin0_w, lin0_b, conv_w, lin1_w, lin1_b):
    src = edge_index[0]
    dst = edge_index[1]
    srcoff = (src[None, :]
              + (jnp.arange(NCHUNK, dtype=jnp.int32) * NPAD)[:, None])
    srcoff = srcoff.reshape(NCHUNK, NS, 2, 1, NB // 2, B)
    dst_r = jnp.broadcast_to(dst.reshape(1, NS, 2, 1, NB // 2, B),
                             (NCHUNK, NS, 2, 1, NB // 2, B))
    edata = jnp.concatenate([srcoff, dst_r], axis=3)

    x0c, x0s = _lin0(x, lin0_w, lin0_b.reshape(1, H))
    w4 = conv_w.reshape(L, NCHUNK, CW, H)

    h = x0c
    for l in range(L - 1):
        beta = float(np.log(THETA / (l + 1) + 1.0))
        aggc = _sc_segsum(h.reshape(NCHUNK * NPAD, CW), x0s, edata)
        h = _tc_layer(aggc, w4[l], beta)

    beta = float(np.log(THETA / L + 1.0))
    aggc = _sc_segsum(h.reshape(NCHUNK * NPAD, CW), x0s, edata)
    return _final_layer(aggc, w4[L - 1],
                        lin1_w.reshape(NCHUNK, CW, C), lin1_b.reshape(1, C),
                        beta)
